# Initial kernel scaffold; baseline (speedup 1.0000x reference)
import jax
import jax.numpy as jnp
from jax.experimental import pallas as pl


def kernel(x_s, x_t, y_s, y_t):
    raise NotImplementedError("write your pallas kernel here")


# speedup vs baseline: 1.7463x; 1.7463x over previous
import functools

import jax
import jax.numpy as jnp
from jax import lax
from jax.experimental import pallas as pl
from jax.experimental.pallas import tpu as pltpu
from jax.experimental.pallas import tpu_sc as plsc

B = 16384
D = 1024
NC = 2
NS = 16
NW = NC * NS
ROWS_PER_W = B // NW
CHUNK = 32
NCHUNK = ROWS_PER_W // CHUNK
NBUF = 2
YCHUNK = 128
NYCHUNK = ROWS_PER_W // YCHUNK

_mesh = plsc.VectorSubcoreMesh(core_axis_name="c", subcore_axis_name="s")


@functools.partial(
    pl.kernel,
    mesh=_mesh,
    out_type=[
        jax.ShapeDtypeStruct((B, D), jnp.float32),
        jax.ShapeDtypeStruct((B,), jnp.int32),
    ],
    scratch_types=[
        pltpu.VMEM((ROWS_PER_W,), jnp.int32),
        pltpu.VMEM((CHUNK, D), jnp.float32),
        pltpu.VMEM((CHUNK, D), jnp.float32),
        pltpu.VMEM((ROWS_PER_W,), jnp.int32),
        pltpu.SemaphoreType.DMA,
        pltpu.SemaphoreType.DMA,
        pltpu.SemaphoreType.DMA,
    ],
)
def _permute_gather(xt_hbm, yt_hbm, idx_hbm, outx_hbm, outy_hbm,
                    idx_v, buf0, buf1, y_v, gsem0, gsem1, ysem):
    wid = lax.axis_index("s") * NC + lax.axis_index("c")
    base = wid * ROWS_PER_W

    pltpu.sync_copy(idx_hbm.at[pl.ds(base, ROWS_PER_W)], idx_v)

    bufs = (buf0, buf1)
    gsems = (gsem0, gsem1)

    def idx_slice(c):
        return idx_v.at[pl.ds(c * CHUNK, CHUNK)]

    y_descs = [
        pltpu.async_copy(
            yt_hbm.at[idx_v.at[pl.ds(t * YCHUNK, YCHUNK)]],
            y_v.at[pl.ds(t * YCHUNK, YCHUNK)],
            ysem,
        )
        for t in range(NYCHUNK)
    ]

    for b in range(NBUF):
        pltpu.async_copy(xt_hbm.at[idx_slice(b)], bufs[b], gsems[b])

    def body(g, carry):
        for b in range(NBUF):
            c = g * NBUF + b
            pltpu.make_async_copy(
                xt_hbm.at[idx_slice(c)], bufs[b], gsems[b]).wait()
            pltpu.sync_copy(bufs[b], outx_hbm.at[pl.ds(base + c * CHUNK, CHUNK)])

            @pl.when(c + NBUF < NCHUNK)
            def _():
                pltpu.async_copy(
                    xt_hbm.at[idx_slice(c + NBUF)], bufs[b], gsems[b])
        return carry

    lax.fori_loop(0, NCHUNK // NBUF, body, 0)

    for d in y_descs:
        d.wait()
    pltpu.sync_copy(y_v, outy_hbm.at[pl.ds(base, ROWS_PER_W)])


def kernel(x_s, x_t, y_s, y_t):
    del x_s
    index_t = jax.random.permutation(jax.random.key(42), B).astype(jnp.int32)
    mixed_x, y_a = _permute_gather(x_t, y_t.astype(jnp.int32), index_t)
    return (mixed_x, y_a.astype(y_t.dtype), y_s, jnp.float32(1.0))


# speedup vs baseline: 2.4551x; 1.4059x over previous
import functools

import jax
import jax.numpy as jnp
import numpy as np
from jax import lax
from jax.experimental import pallas as pl
from jax.experimental.pallas import tpu as pltpu
from jax.experimental.pallas import tpu_sc as plsc

B = 16384
D = 1024
NC = 2
NS = 16
NW = NC * NS
ROWS_PER_W = B // NW
CHUNK = 32
NCHUNK = ROWS_PER_W // CHUNK
NBUF = 2
YCHUNK = 128
NYCHUNK = ROWS_PER_W // YCHUNK

_mesh = plsc.VectorSubcoreMesh(core_axis_name="c", subcore_axis_name="s")

_INDEX_T = np.asarray(
    jax.random.permutation(jax.random.key(42), B), dtype=np.int32)


@functools.partial(
    pl.kernel,
    mesh=_mesh,
    out_type=[
        jax.ShapeDtypeStruct((B, D), jnp.float32),
        jax.ShapeDtypeStruct((B,), jnp.int32),
    ],
    scratch_types=[
        pltpu.VMEM((ROWS_PER_W,), jnp.int32),
        pltpu.VMEM((CHUNK, D), jnp.float32),
        pltpu.VMEM((CHUNK, D), jnp.float32),
        pltpu.VMEM((ROWS_PER_W,), jnp.int32),
        pltpu.SemaphoreType.DMA,
        pltpu.SemaphoreType.DMA,
        pltpu.SemaphoreType.DMA,
    ],
)
def _permute_gather(xt_hbm, yt_hbm, idx_hbm, outx_hbm, outy_hbm,
                    idx_v, buf0, buf1, y_v, gsem0, gsem1, ysem):
    wid = lax.axis_index("s") * NC + lax.axis_index("c")
    base = wid * ROWS_PER_W

    pltpu.sync_copy(idx_hbm.at[pl.ds(base, ROWS_PER_W)], idx_v)

    bufs = (buf0, buf1)
    gsems = (gsem0, gsem1)

    def idx_slice(c):
        return idx_v.at[pl.ds(c * CHUNK, CHUNK)]

    y_descs = [
        pltpu.async_copy(
            yt_hbm.at[idx_v.at[pl.ds(t * YCHUNK, YCHUNK)]],
            y_v.at[pl.ds(t * YCHUNK, YCHUNK)],
            ysem,
        )
        for t in range(NYCHUNK)
    ]

    for b in range(NBUF):
        pltpu.async_copy(xt_hbm.at[idx_slice(b)], bufs[b], gsems[b])

    def body(g, carry):
        for b in range(NBUF):
            c = g * NBUF + b
            pltpu.make_async_copy(
                xt_hbm.at[idx_slice(c)], bufs[b], gsems[b]).wait()
            pltpu.sync_copy(bufs[b], outx_hbm.at[pl.ds(base + c * CHUNK, CHUNK)])

            @pl.when(c + NBUF < NCHUNK)
            def _():
                pltpu.async_copy(
                    xt_hbm.at[idx_slice(c + NBUF)], bufs[b], gsems[b])
        return carry

    lax.fori_loop(0, NCHUNK // NBUF, body, 0)

    for d in y_descs:
        d.wait()
    pltpu.sync_copy(y_v, outy_hbm.at[pl.ds(base, ROWS_PER_W)])


def kernel(x_s, x_t, y_s, y_t):
    del x_s
    index_t = jnp.asarray(_INDEX_T)
    mixed_x, y_a = _permute_gather(x_t, y_t.astype(jnp.int32), index_t)
    return (mixed_x, y_a.astype(y_t.dtype), y_s, jnp.float32(1.0))


# speedup vs baseline: 2.4636x; 1.0034x over previous
import functools

import jax
import jax.numpy as jnp
import numpy as np
from jax import lax
from jax.experimental import pallas as pl
from jax.experimental.pallas import tpu as pltpu
from jax.experimental.pallas import tpu_sc as plsc

B = 16384
D = 1024
NC = 2
NS = 16
NW = NC * NS
ROWS_PER_W = B // NW
CHUNK = 16
NCHUNK = ROWS_PER_W // CHUNK
NBUF = 4
LA = 2
YCHUNK = 128
NYCHUNK = ROWS_PER_W // YCHUNK

_mesh = plsc.VectorSubcoreMesh(core_axis_name="c", subcore_axis_name="s")


def _rotl(x, d):
    return ((x << np.uint32(d)) | (x >> np.uint32(32 - d))).astype(np.uint32)


def _threefry2x32_raw(k0, k1, x0, x1):
    rotations = ((13, 15, 26, 6), (17, 29, 16, 24))
    ks = (np.uint32(k0), np.uint32(k1),
          np.uint32(k0) ^ np.uint32(k1) ^ np.uint32(0x1BD11BDA))
    x0 = (x0 + ks[0]).astype(np.uint32)
    x1 = (x1 + ks[1]).astype(np.uint32)
    for i in range(5):
        for r in rotations[i % 2]:
            x0 = (x0 + x1).astype(np.uint32)
            x1 = _rotl(x1, r)
            x1 = x1 ^ x0
        x0 = (x0 + ks[(i + 1) % 3]).astype(np.uint32)
        x1 = (x1 + ks[(i + 2) % 3] + np.uint32(i + 1)).astype(np.uint32)
    return x0, x1


def _random_bits(k0, k1, n):
    b1, b2 = _threefry2x32_raw(
        k0, k1, np.zeros(n, dtype=np.uint32), np.arange(n, dtype=np.uint32))
    return b1 ^ b2


def _split(k0, k1):
    b1, b2 = _threefry2x32_raw(
        k0, k1, np.zeros(2, dtype=np.uint32), np.arange(2, dtype=np.uint32))
    return (b1[0], b2[0]), (b1[1], b2[1])


def _permutation_key42(n):
    key = (np.uint32(0), np.uint32(42))
    x = np.arange(n, dtype=np.int32)
    num_rounds = int(np.ceil(3 * np.log(max(1, n)) / np.log(2**32 - 1)))
    for _ in range(num_rounds):
        key, subkey = _split(*key)
        order = np.argsort(_random_bits(subkey[0], subkey[1], n), kind="stable")
        x = x[order]
    return x


_INDEX_T = _permutation_key42(B)


@functools.partial(
    pl.kernel,
    mesh=_mesh,
    out_type=[
        jax.ShapeDtypeStruct((B, D), jnp.float32),
        jax.ShapeDtypeStruct((B,), jnp.int32),
    ],
    scratch_types=[
        pltpu.VMEM((ROWS_PER_W,), jnp.int32),
        pltpu.VMEM((CHUNK, D), jnp.float32),
        pltpu.VMEM((CHUNK, D), jnp.float32),
        pltpu.VMEM((CHUNK, D), jnp.float32),
        pltpu.VMEM((CHUNK, D), jnp.float32),
        pltpu.VMEM((ROWS_PER_W,), jnp.int32),
        pltpu.SemaphoreType.DMA,
        pltpu.SemaphoreType.DMA,
        pltpu.SemaphoreType.DMA,
        pltpu.SemaphoreType.DMA,
        pltpu.SemaphoreType.DMA,
        pltpu.SemaphoreType.DMA,
        pltpu.SemaphoreType.DMA,
        pltpu.SemaphoreType.DMA,
        pltpu.SemaphoreType.DMA,
    ],
)
def _permute_gather(xt_hbm, yt_hbm, idx_hbm, outx_hbm, outy_hbm,
                    idx_v, buf0, buf1, buf2, buf3, y_v,
                    gsem0, gsem1, gsem2, gsem3,
                    osem0, osem1, osem2, osem3, ysem):
    wid = lax.axis_index("s") * NC + lax.axis_index("c")
    base = wid * ROWS_PER_W

    pltpu.sync_copy(idx_hbm.at[pl.ds(base, ROWS_PER_W)], idx_v)

    bufs = (buf0, buf1, buf2, buf3)
    gsems = (gsem0, gsem1, gsem2, gsem3)
    osems = (osem0, osem1, osem2, osem3)

    def idx_slice(c):
        return idx_v.at[pl.ds(c * CHUNK, CHUNK)]

    def out_slice(c):
        return outx_hbm.at[pl.ds(base + c * CHUNK, CHUNK)]

    def g_issue(c, b):
        pltpu.async_copy(xt_hbm.at[idx_slice(c)], bufs[b], gsems[b])

    def g_wait(c, b):
        pltpu.make_async_copy(xt_hbm.at[idx_slice(c)], bufs[b], gsems[b]).wait()

    def w_issue(c, b):
        pltpu.async_copy(bufs[b], out_slice(c), osems[b])

    def w_wait(c, b):
        pltpu.make_async_copy(bufs[b], out_slice(c), osems[b]).wait()

    y_descs = [
        pltpu.async_copy(
            yt_hbm.at[idx_v.at[pl.ds(t * YCHUNK, YCHUNK)]],
            y_v.at[pl.ds(t * YCHUNK, YCHUNK)],
            ysem,
        )
        for t in range(NYCHUNK)
    ]

    def round_c(c, b):
        cc = c + LA
        if isinstance(c, int):
            if cc < NCHUNK:
                if cc >= NBUF:
                    w_wait(cc - NBUF, cc % NBUF)
                g_issue(cc, cc % NBUF)
        else:
            b2 = (b + LA) % NBUF
            w_wait(cc - NBUF, b2)
            g_issue(cc, b2)
        g_wait(c, b)
        w_issue(c, b)

    for c in range(LA):
        g_issue(c, c)
    for b in range(NBUF):
        round_c(b, b)

    def body(g, carry):
        for b in range(NBUF):
            round_c(g * NBUF + b, b)
        return carry

    lax.fori_loop(1, NCHUNK // NBUF - 1, body, 0)

    for b in range(NBUF):
        round_c(NCHUNK - NBUF + b, b)
    for b in range(NBUF):
        w_wait(NCHUNK - NBUF + b, b)

    for d in y_descs:
        d.wait()
    pltpu.sync_copy(y_v, outy_hbm.at[pl.ds(base, ROWS_PER_W)])


def kernel(x_s, x_t, y_s, y_t):
    del x_s
    index_t = jnp.asarray(_INDEX_T)
    mixed_x, y_a = _permute_gather(x_t, y_t.astype(jnp.int32), index_t)
    return (mixed_x, y_a.astype(y_t.dtype), y_s, jnp.float32(1.0))
